# baseline (speedup 1.0000x reference)
import jax
import jax.numpy as jnp
from jax.experimental import pallas as pl
from jax.experimental.pallas import tpu as pltpu
from jax.experimental.pallas import tpu_sc as plsc

_BLOCK_ROWS = 16
_LANES = 16


def kernel(input, pe_table):
    length = input.shape[1]
    max_len, d = pe_table.shape
    mesh = plsc.VectorSubcoreMesh(core_axis_name="core", subcore_axis_name="subcore")

    @pl.kernel(out_type=jax.ShapeDtypeStruct((length, d), pe_table.dtype),
               mesh=mesh)
    def sc_gather_rows(pe_hbm, o_hbm):
        def body(in_vmem, out_vmem):
            @pl.loop(0, _BLOCK_ROWS)
            def _(r):
                for c in range(0, d, _LANES):
                    out_vmem.at[r, pl.ds(c, _LANES)][...] = (
                        in_vmem.at[r, pl.ds(c, _LANES)][...])

        pltpu.emit_pipeline(
            body,
            grid=(length // _BLOCK_ROWS,),
            in_specs=[pl.BlockSpec((_BLOCK_ROWS, d), lambda i: (i, 0))],
            out_specs=[pl.BlockSpec((_BLOCK_ROWS, d), lambda i: (i, 0))],
            core_axis_name=("core", "subcore"),
            dimension_semantics=(pltpu.PARALLEL,),
        )(pe_hbm, o_hbm)

    return sc_gather_rows(pe_table)[None]


# speedup vs baseline: 1.7693x; 1.7693x over previous
import jax
import jax.numpy as jnp
from jax.experimental import pallas as pl
from jax.experimental.pallas import tpu as pltpu
from jax.experimental.pallas import tpu_sc as plsc

_NUM_CORES = 2
_NUM_SUBCORES = 16
_CHUNK_ROWS = 32


def kernel(input, pe_table):
    length = input.shape[1]
    max_len, d = pe_table.shape
    units = _NUM_CORES * _NUM_SUBCORES
    rows_per_unit = length // units
    nblk = rows_per_unit // _CHUNK_ROWS

    mesh = plsc.VectorSubcoreMesh(core_axis_name="core", subcore_axis_name="subcore")

    @pl.kernel(out_type=jax.ShapeDtypeStruct((length, d), pe_table.dtype),
               mesh=mesh,
               scratch_types=[pltpu.VMEM((2, _CHUNK_ROWS, d), pe_table.dtype),
                              pltpu.SemaphoreType.DMA((2,))])
    def sc_gather_rows(pe_hbm, o_hbm, buf, in_sem):
        core = jax.lax.axis_index("core")
        sub = jax.lax.axis_index("subcore")
        base = (core * _NUM_SUBCORES + sub) * rows_per_unit

        def rd(i):
            s = i % 2
            return pltpu.make_async_copy(
                pe_hbm.at[pl.ds(base + i * _CHUNK_ROWS, _CHUNK_ROWS)],
                buf.at[s], in_sem.at[s])

        rd(0).start()
        for i in range(nblk):
            rd(i).wait()
            if i + 1 < nblk:
                rd(i + 1).start()
            pltpu.sync_copy(
                buf.at[i % 2],
                o_hbm.at[pl.ds(base + i * _CHUNK_ROWS, _CHUNK_ROWS)])

    return sc_gather_rows(pe_table)[None]
